# Initial kernel scaffold; baseline (speedup 1.0000x reference)
import jax
import jax.numpy as jnp
from jax.experimental import pallas as pl


def kernel(x, edge_index, batch, W1, as1, ad1, b1, W2, as2, ad2, b2, W3, as3, ad3, b3, Wp, bp):
    raise NotImplementedError("write your pallas kernel here")


# speedup vs baseline: 8.1713x; 8.1713x over previous
import functools

import jax
import jax.numpy as jnp
from jax import lax
from jax.experimental import pallas as pl
from jax.experimental.pallas import tpu as pltpu
from jax.experimental.pallas import tpu_sc as plsc

N = 10000
D = 128
H = 128
C = 32
G = 64

NC = 2
NS = 16
LANE = 16

NHALF = N // NC
TRASH = NHALF
HP = H + 16
NB = 1000
NGRID = N // NB

EPS = 1e-16


def _tc_first(x, W, a2):
    def body(x_ref, w_ref, a_ref, h_ref, sa_ref, da_ref):
        h = jnp.dot(x_ref[...], w_ref[...], preferred_element_type=jnp.float32)
        h_ref[...] = h
        sada = jnp.dot(h, a_ref[...], preferred_element_type=jnp.float32)
        sa_ref[...] = sada[:, 0:1]
        da_ref[...] = sada[:, 1:2]

    return pl.pallas_call(
        body,
        grid=(NGRID,),
        in_specs=[
            pl.BlockSpec((NB, D), lambda k: (k, 0)),
            pl.BlockSpec((D, H), lambda k: (0, 0)),
            pl.BlockSpec((H, 2), lambda k: (0, 0)),
        ],
        out_specs=[
            pl.BlockSpec((NB, H), lambda k: (k, 0)),
            pl.BlockSpec((NB, 1), lambda k: (k, 0)),
            pl.BlockSpec((NB, 1), lambda k: (k, 0)),
        ],
        out_shape=[
            jax.ShapeDtypeStruct((N, H), jnp.float32),
            jax.ShapeDtypeStruct((N, 1), jnp.float32),
            jax.ShapeDtypeStruct((N, 1), jnp.float32),
        ],
    )(x, W, a2)


def _tc_mid(acc, b2d, W, a2):
    def body(acc_ref, b_ref, w_ref, a_ref, h_ref, sa_ref, da_ref):
        a0 = acc_ref[...]
        num = a0[:, :H]
        den = a0[:, H:H + 1] + EPS
        feat = jnp.maximum(num / den + b_ref[...], 0.0)
        h = jnp.dot(feat, w_ref[...], preferred_element_type=jnp.float32)
        h_ref[...] = h
        sada = jnp.dot(h, a_ref[...], preferred_element_type=jnp.float32)
        sa_ref[...] = sada[:, 0:1]
        da_ref[...] = sada[:, 1:2]

    return pl.pallas_call(
        body,
        grid=(NGRID,),
        in_specs=[
            pl.BlockSpec((NB, HP), lambda k: (k, 0)),
            pl.BlockSpec((1, H), lambda k: (0, 0)),
            pl.BlockSpec((D, H), lambda k: (0, 0)),
            pl.BlockSpec((H, 2), lambda k: (0, 0)),
        ],
        out_specs=[
            pl.BlockSpec((NB, H), lambda k: (k, 0)),
            pl.BlockSpec((NB, 1), lambda k: (k, 0)),
            pl.BlockSpec((NB, 1), lambda k: (k, 0)),
        ],
        out_shape=[
            jax.ShapeDtypeStruct((N, H), jnp.float32),
            jax.ShapeDtypeStruct((N, 1), jnp.float32),
            jax.ShapeDtypeStruct((N, 1), jnp.float32),
        ],
    )(acc, b2d, W, a2)


def _tc_readout(acc, b2d, batch3d, batchm13d, Wp, bp2d):
    def body(acc_ref, b_ref, bat_ref, batm1_ref, wp_ref, bp_ref, out_ref,
             hsel_ref, pres_ref):
        k = pl.program_id(0)

        a0 = acc_ref[...]
        num = a0[:, :H]
        den = a0[:, H:H + 1] + EPS
        h3 = num / den + b_ref[...]

        bat = bat_ref[0]
        batm1 = batm1_ref[0]
        col = lax.broadcasted_iota(jnp.int32, (1, NB), 1) + k * NB
        first = jnp.logical_or(col == 0, bat != batm1)
        gids = lax.broadcasted_iota(jnp.int32, (G, NB), 0)
        onehot = jnp.where(
            jnp.logical_and(bat == gids, first), 1.0, 0.0
        ).astype(jnp.float32)

        part = jnp.dot(onehot, h3, preferred_element_type=jnp.float32)
        pcnt = jnp.sum(onehot, axis=1, keepdims=True)

        @pl.when(k == 0)
        def _():
            hsel_ref[...] = part
            pres_ref[...] = pcnt

        @pl.when(k > 0)
        def _():
            hsel_ref[...] = hsel_ref[...] + part
            pres_ref[...] = pres_ref[...] + pcnt

        @pl.when(k == NGRID - 1)
        def _():
            lastrow = h3[NB - 1:NB, :]
            hsel = hsel_ref[...] + (1.0 - pres_ref[...]) * lastrow
            out_ref[...] = (
                jnp.dot(hsel, wp_ref[...], preferred_element_type=jnp.float32)
                + bp_ref[...]
            )

    return pl.pallas_call(
        body,
        grid=(NGRID,),
        in_specs=[
            pl.BlockSpec((NB, HP), lambda k: (k, 0)),
            pl.BlockSpec((1, H), lambda k: (0, 0)),
            pl.BlockSpec((1, 1, NB), lambda k: (k, 0, 0)),
            pl.BlockSpec((1, 1, NB), lambda k: (k, 0, 0)),
            pl.BlockSpec((H, C), lambda k: (0, 0)),
            pl.BlockSpec((1, C), lambda k: (0, 0)),
        ],
        out_specs=pl.BlockSpec((G, C), lambda k: (0, 0)),
        out_shape=jax.ShapeDtypeStruct((G, C), jnp.float32),
        scratch_shapes=[
            pltpu.VMEM((G, H), jnp.float32),
            pltpu.VMEM((G, 1), jnp.float32),
        ],
    )(acc, b2d, batch3d, batchm13d, Wp, bp2d)


def _sc_edge_pass(src_pad, dst_pad, h, sa, da, e_real, e_tot):
    per_tile = e_tot // NS
    n_chunks = per_tile // 128
    RCH = 40
    n_rch = NHALF // RCH
    mesh = plsc.VectorSubcoreMesh(core_axis_name="c", subcore_axis_name="s")

    @functools.partial(
        pl.kernel,
        out_type=jax.ShapeDtypeStruct((N, HP), jnp.float32),
        mesh=mesh,
        compiler_params=pltpu.CompilerParams(
            needs_layout_passes=False, use_tc_tiling_on_sc=False
        ),
        scratch_types=[
            pltpu.VMEM((N,), jnp.float32),
            pltpu.VMEM((N,), jnp.float32),
            pltpu.VMEM((128,), jnp.int32),
            pltpu.VMEM((128,), jnp.int32),
            pltpu.VMEM((128,), jnp.float32),
            pltpu.VMEM((128, H), jnp.float32),
            pltpu.VMEM((128, HP), jnp.float32),
            pltpu.VMEM_SHARED((NHALF + 8, HP), jnp.float32),
            pltpu.SemaphoreType.DMA,
        ],
    )
    def k(src_hbm, dst_hbm, h_hbm, sa_hbm, da_hbm, out_hbm,
          sa_v, da_v, srcb, dstb, ev, grow, wrow, acc_sh, sem):
        c = lax.axis_index("c")
        s = lax.axis_index("s")
        row0 = c * NHALF

        zed = jnp.zeros((LANE,), jnp.float32)

        def zrow(r, _):
            for j in range(HP // LANE):
                wrow[r, pl.ds(j * LANE, LANE)] = zed
            return 0

        lax.fori_loop(0, 128, zrow, 0)
        for i in range((n_rch + NS - 1) // NS):
            idx = s + NS * i

            @pl.when(idx < n_rch)
            def _():
                pltpu.sync_copy(
                    wrow.at[pl.ds(0, RCH)],
                    acc_sh.at[pl.ds(idx * RCH, RCH)],
                )

        @pl.when(s == 0)
        def _():
            pltpu.sync_copy(wrow.at[pl.ds(0, 8)], acc_sh.at[pl.ds(NHALF, 8)])

        pltpu.sync_copy(sa_hbm, sa_v)
        pltpu.sync_copy(da_hbm, da_v)
        plsc.subcore_barrier()

        def chunk_body(ch, _):
            base = s * per_tile + ch * 128
            pltpu.sync_copy(src_hbm.at[pl.ds(base, 128)], srcb)
            pltpu.sync_copy(dst_hbm.at[pl.ds(base, 128)], dstb)
            for j in range(8):
                sv = srcb[pl.ds(j * LANE, LANE)]
                dv = dstb[pl.ds(j * LANE, LANE)]
                sav = plsc.load_gather(sa_v, [sv])
                dav = plsc.load_gather(da_v, [dv])
                xv = sav + dav
                lv = jnp.where(xv >= 0.0, xv, 0.2 * xv)
                eid = base + j * LANE + lax.iota(jnp.int32, LANE)
                evv = jnp.where(eid < e_real, jnp.exp(lv), 0.0)
                ev[pl.ds(j * LANE, LANE)] = evv
                dloc = dv - row0
                owned = jnp.logical_and(dloc >= 0, dloc < NHALF)
                dstb[pl.ds(j * LANE, LANE)] = jnp.where(owned, dloc, TRASH)
            pltpu.async_copy(h_hbm.at[srcb], grow, sem).wait()

            def row_group(g, _):
                evv = ev[pl.ds(g * LANE, LANE)]
                for i in range(LANE):
                    er = jnp.full((LANE,), evv[i], jnp.float32)
                    r = g * LANE + i
                    for j in range(H // LANE):
                        wrow[r, pl.ds(j * LANE, LANE)] = (
                            grow[r, pl.ds(j * LANE, LANE)] * er
                        )
                    wrow[r, pl.ds(H, LANE)] = er
                return 0

            lax.fori_loop(0, 8, row_group, 0)
            pltpu.sync_copy(wrow, acc_sh.at[dstb], add=True)
            return 0

        lax.fori_loop(0, n_chunks, chunk_body, 0)
        plsc.subcore_barrier()

        for i in range((n_rch + NS - 1) // NS):
            idx = s + NS * i

            @pl.when(idx < n_rch)
            def _():
                r0 = idx * RCH
                pltpu.sync_copy(acc_sh.at[pl.ds(r0, RCH)], wrow.at[pl.ds(0, RCH)])
                pltpu.sync_copy(
                    wrow.at[pl.ds(0, RCH)], out_hbm.at[pl.ds(row0 + r0, RCH)]
                )

    return k(src_pad, dst_pad, h, sa, da)


def kernel(x, edge_index, batch, W1, as1, ad1, b1, W2, as2, ad2, b2,
           W3, as3, ad3, b3, Wp, bp):
    E = edge_index.shape[1]
    e_real = E + N
    e_tot = ((e_real + NS * 128 - 1) // (NS * 128)) * (NS * 128)
    pad = e_tot - e_real

    loops = jnp.arange(N, dtype=jnp.int32)
    zpad = jnp.zeros((pad,), jnp.int32)
    src = jnp.concatenate([edge_index[0].astype(jnp.int32), loops, zpad])
    dst = jnp.concatenate([edge_index[1].astype(jnp.int32), loops, zpad])

    a21 = jnp.stack([as1, ad1], axis=1)
    a22 = jnp.stack([as2, ad2], axis=1)
    a23 = jnp.stack([as3, ad3], axis=1)

    h1, sa1, da1 = _tc_first(x.astype(jnp.float32), W1, a21)
    acc1 = _sc_edge_pass(src, dst, h1, sa1.reshape(N), da1.reshape(N),
                         e_real, e_tot)
    h2, sa2, da2 = _tc_mid(acc1, b1.reshape(1, H), W2, a22)
    acc2 = _sc_edge_pass(src, dst, h2, sa2.reshape(N), da2.reshape(N),
                         e_real, e_tot)
    h3, sa3, da3 = _tc_mid(acc2, b2.reshape(1, H), W3, a23)
    acc3 = _sc_edge_pass(src, dst, h3, sa3.reshape(N), da3.reshape(N),
                         e_real, e_tot)

    batf = batch.astype(jnp.int32)
    batm1f = jnp.concatenate([batf[:1], batf[:-1]])
    batch3d = batf.reshape(NGRID, 1, NB)
    batchm13d = batm1f.reshape(NGRID, 1, NB)
    logits = _tc_readout(acc3, b3.reshape(1, H), batch3d, batchm13d,
                         Wp, bp.reshape(1, C))
    return logits
